# baseline (speedup 1.0000x reference)
import functools

import jax
import jax.numpy as jnp
from jax import lax
from jax.experimental import pallas as pl
from jax.experimental.pallas import tpu as pltpu
from jax.experimental.pallas import tpu_sc as plsc

HIDDEN = 768
MAX_POS = 77
N_ROWS = 4096 * 77
NC, NS, L = 2, 16, 16
NW = NC * NS
ROWS_PER_W = N_ROWS // NW
CHUNK = 32
STEPS = ROWS_PER_W // CHUNK
VPR = HIDDEN // L

_mesh = plsc.VectorSubcoreMesh(core_axis_name="c", subcore_axis_name="s")


@functools.partial(
    pl.kernel,
    out_type=jax.ShapeDtypeStruct((N_ROWS, HIDDEN), jnp.float32),
    mesh=_mesh,
    scratch_types=[
        pltpu.VMEM((ROWS_PER_W,), jnp.int32),
        pltpu.VMEM((MAX_POS, HIDDEN), jnp.float32),
        pltpu.VMEM((CHUNK, HIDDEN), jnp.float32),
        pltpu.VMEM((CHUNK, HIDDEN), jnp.float32),
        pltpu.SemaphoreType.DMA,
        pltpu.SemaphoreType.DMA,
        pltpu.SemaphoreType.DMA,
        pltpu.SemaphoreType.DMA,
    ],
)
def _emb_kernel(ids_hbm, tok_hbm, pos_hbm, out_hbm,
                idx_v, pos_v, buf0, buf1, sg0, sg1, ss0, ss1):
  wid = lax.axis_index("s") * NC + lax.axis_index("c")
  w_base = wid * ROWS_PER_W
  bufs = (buf0, buf1)
  gsems = (sg0, sg1)
  ssems = (ss0, ss1)

  pltpu.sync_copy(ids_hbm.at[pl.ds(w_base, ROWS_PER_W)], idx_v)
  pltpu.sync_copy(pos_hbm, pos_v)

  def start_gather(c, b):
    pltpu.async_copy(tok_hbm.at[idx_v.at[pl.ds(c * CHUNK, CHUNK)]],
                     bufs[b], gsems[b])

  def wait_scatter(b):
    pltpu.make_async_copy(bufs[b], out_hbm.at[pl.ds(0, CHUNK)],
                          ssems[b]).wait()

  def process(c, b):
    pltpu.make_async_copy(tok_hbm.at[idx_v.at[pl.ds(0, CHUNK)]],
                          bufs[b], gsems[b]).wait()
    row_base = w_base + c * CHUNK

    def add_row(r, carry):
      p = (row_base + r) % MAX_POS
      for k in range(VPR):
        sl = pl.ds(k * L, L)
        bufs[b][r, sl] = bufs[b][r, sl] + pos_v[p, sl]
      return carry

    lax.fori_loop(0, CHUNK, add_row, 0)
    pltpu.async_copy(bufs[b], out_hbm.at[pl.ds(row_base, CHUNK)], ssems[b])

  start_gather(0, 0)

  def step(c, b):
    @pl.when(c >= 2)
    def _():
      wait_scatter(b)

    @pl.when(c < STEPS)
    def _():
      start_gather(c, b)

    process(c - 1, 1 - b)

  def pair(c2, carry):
    step(2 * c2 + 1, 1)
    step(2 * c2 + 2, 0)
    return carry

  lax.fori_loop(0, STEPS // 2, pair, 0)

  wait_scatter(1)


def kernel(input_ids, token_table, pos_table):
  Bn, Tn = input_ids.shape
  ids = input_ids.reshape(-1).astype(jnp.int32)
  out = _emb_kernel(ids, token_table, pos_table)
  return out.reshape(Bn, Tn, HIDDEN)


# speedup vs baseline: 1.6085x; 1.6085x over previous
import functools

import jax
import jax.numpy as jnp
from jax import lax
from jax.experimental import pallas as pl
from jax.experimental.pallas import tpu as pltpu
from jax.experimental.pallas import tpu_sc as plsc

HIDDEN = 768
MAX_POS = 77
N_ROWS = 4096 * 77
NC, NS, L = 2, 16, 16
NW = NC * NS
ROWS_PER_W = N_ROWS // NW
CHUNK = 32
STEPS = ROWS_PER_W // CHUNK
VPR = HIDDEN // L
GRP = 16

_mesh = plsc.VectorSubcoreMesh(core_axis_name="c", subcore_axis_name="s")


@functools.partial(
    pl.kernel,
    out_type=jax.ShapeDtypeStruct((N_ROWS, HIDDEN), jnp.float32),
    mesh=_mesh,
    scratch_types=[
        pltpu.VMEM((ROWS_PER_W,), jnp.int32),
        pltpu.VMEM((MAX_POS, HIDDEN), jnp.float32),
        pltpu.VMEM((CHUNK, HIDDEN), jnp.float32),
        pltpu.VMEM((CHUNK, HIDDEN), jnp.float32),
        pltpu.SemaphoreType.DMA,
        pltpu.SemaphoreType.DMA,
        pltpu.SemaphoreType.DMA,
        pltpu.SemaphoreType.DMA,
    ],
)
def _emb_kernel(ids_hbm, tok_hbm, pos_hbm, out_hbm,
                idx_v, pos_v, buf0, buf1, sg0, sg1, ss0, ss1):
  wid = lax.axis_index("s") * NC + lax.axis_index("c")
  w_base = wid * ROWS_PER_W
  bufs = (buf0, buf1)
  gsems = (sg0, sg1)
  ssems = (ss0, ss1)

  pltpu.sync_copy(ids_hbm.at[pl.ds(w_base, ROWS_PER_W)], idx_v)
  pltpu.sync_copy(pos_hbm, pos_v)

  def start_gather(c, b):
    pltpu.async_copy(tok_hbm.at[idx_v.at[pl.ds(c * CHUNK, CHUNK)]],
                     bufs[b], gsems[b])

  def wait_scatter(b):
    pltpu.make_async_copy(bufs[b], out_hbm.at[pl.ds(0, CHUNK)],
                          ssems[b]).wait()

  def process(c, b):
    pltpu.make_async_copy(tok_hbm.at[idx_v.at[pl.ds(0, CHUNK)]],
                          bufs[b], gsems[b]).wait()
    row_base = w_base + c * CHUNK

    def add_row(r, p):
      for g in range(VPR // GRP):
        tv = [bufs[b][r, pl.ds((g * GRP + k) * L, L)] for k in range(GRP)]
        pv = [pos_v[p, pl.ds((g * GRP + k) * L, L)] for k in range(GRP)]
        for k in range(GRP):
          bufs[b][r, pl.ds((g * GRP + k) * L, L)] = tv[k] + pv[k]
      p = p + 1
      return jnp.where(p == MAX_POS, 0, p)

    lax.fori_loop(0, CHUNK, add_row, (row_base % MAX_POS).astype(jnp.int32))
    pltpu.async_copy(bufs[b], out_hbm.at[pl.ds(row_base, CHUNK)], ssems[b])

  start_gather(0, 0)

  def step(c, b):
    @pl.when(c >= 2)
    def _():
      wait_scatter(b)

    @pl.when(c < STEPS)
    def _():
      start_gather(c, b)

    process(c - 1, 1 - b)

  def pair(c2, carry):
    step(2 * c2 + 1, 1)
    step(2 * c2 + 2, 0)
    return carry

  lax.fori_loop(0, STEPS // 2, pair, 0)

  wait_scatter(1)


def kernel(input_ids, token_table, pos_table):
  Bn, Tn = input_ids.shape
  ids = input_ids.reshape(-1).astype(jnp.int32)
  out = _emb_kernel(ids, token_table, pos_table)
  return out.reshape(Bn, Tn, HIDDEN)


# speedup vs baseline: 1.9349x; 1.2029x over previous
import functools

import jax
import jax.numpy as jnp
from jax import lax
from jax.experimental import pallas as pl
from jax.experimental.pallas import tpu as pltpu
from jax.experimental.pallas import tpu_sc as plsc

B = 4096
HIDDEN = 768
MAX_POS = 77
NC, NS, L = 2, 16, 16
NW = NC * NS
SEQ_PER_W = B // NW
VPR = HIDDEN // L
GRP = 16

_mesh = plsc.VectorSubcoreMesh(core_axis_name="c", subcore_axis_name="s")


@functools.partial(
    pl.kernel,
    out_type=jax.ShapeDtypeStruct((B, MAX_POS, HIDDEN), jnp.float32),
    mesh=_mesh,
    scratch_types=[
        pltpu.VMEM((MAX_POS,), jnp.int32),
        pltpu.VMEM((MAX_POS, HIDDEN), jnp.float32),
        pltpu.VMEM((MAX_POS, HIDDEN), jnp.float32),
        pltpu.SemaphoreType.DMA,
    ],
)
def _emb_kernel(ids_hbm, tok_hbm, pos_hbm, out_hbm,
                idx_v, pos_v, buf, sem):
  wid = lax.axis_index("s") * NC + lax.axis_index("c")
  b_base = wid * SEQ_PER_W

  pltpu.sync_copy(pos_hbm, pos_v)

  def seq_body(s, carry):
    pltpu.sync_copy(ids_hbm.at[b_base + s], idx_v)
    pltpu.async_copy(tok_hbm.at[idx_v], buf, sem).wait()

    def add_row(r, carry2):
      for g in range(VPR // GRP):
        tv = [buf[r, pl.ds((g * GRP + k) * L, L)] for k in range(GRP)]
        pv = [pos_v[r, pl.ds((g * GRP + k) * L, L)] for k in range(GRP)]
        for k in range(GRP):
          buf[r, pl.ds((g * GRP + k) * L, L)] = tv[k] + pv[k]
      return carry2

    lax.fori_loop(0, MAX_POS, add_row, 0)
    pltpu.sync_copy(buf, out_hbm.at[b_base + s])
    return carry

  lax.fori_loop(0, SEQ_PER_W, seq_body, 0)


def kernel(input_ids, token_table, pos_table):
  ids = input_ids.astype(jnp.int32)
  return _emb_kernel(ids, token_table, pos_table)
